# Initial kernel scaffold; baseline (speedup 1.0000x reference)
import functools

import jax
import jax.numpy as jnp
from jax import lax
from jax.experimental import pallas as pl
from jax.experimental.pallas import tpu as pltpu
from jax.experimental.pallas import tpu_sc as plsc

NC = 2
NS = 16
NW = NC * NS
LANES = 16


def _prep_body(x_ref, wp_ref, bp_ref, wt_ref, bt_ref, y_ref):
    xb = x_ref[...]
    s = jnp.sum(xb * wt_ref[...], axis=1, keepdims=True) + bt_ref[...]
    t = jax.nn.sigmoid(s)
    xp = jnp.dot(xb, wp_ref[...], preferred_element_type=jnp.float32)
    y_ref[...] = (xp + bp_ref[...]) * t


def _edge_weight_body(blk_ref, dist_ref, a_ref):
    a_ref[...] = 1.0 / (1.0 + blk_ref[...] + 0.1 * dist_ref[...])


def _final_add_body(x_ref, p0_ref, p1_ref, o_ref):
    o_ref[...] = x_ref[...] + p0_ref[...] + p1_ref[...]


def _broadcast_lane(vec, lane):
    idx = jnp.full((LANES, 1), lane, dtype=jnp.int32)
    dnums = lax.GatherDimensionNumbers(
        offset_dims=(), collapsed_slice_dims=(0,), start_index_map=(0,))
    return lax.gather(vec, idx, dnums, slice_sizes=(1,),
                      mode=lax.GatherScatterMode.PROMISE_IN_BOUNDS)


def _make_sc_scatter(n_nodes, d, e_pad_rows, chunk_rows):
    rows_per_worker = e_pad_rows // NW
    n_chunks = rows_per_worker // chunk_rows
    chunk_edges = chunk_rows * 128
    nodes_per_sub = n_nodes // NS
    zrows = 125
    mesh = plsc.VectorSubcoreMesh(core_axis_name="c", subcore_axis_name="s")

    @functools.partial(
        pl.kernel,
        mesh=mesh,
        out_type=jax.ShapeDtypeStruct((NC, n_nodes, d), jnp.float32),
        scratch_types=[
            pltpu.VMEM((chunk_rows, 128), jnp.int32),
            pltpu.VMEM((chunk_rows, 128), jnp.int32),
            pltpu.VMEM((chunk_rows, 128), jnp.float32),
            pltpu.VMEM((chunk_edges, d), jnp.float32),
            pltpu.VMEM((125, d), jnp.float32),
            pltpu.VMEM_SHARED((n_nodes, d), jnp.float32),
            pltpu.SemaphoreType.DMA,
        ],
    )
    def sc_kernel(y_hbm, row_hbm, col_hbm, a_hbm, out_hbm,
                  row_v, col_v, a_v, rows_v, zero_v, acc, sem):
        cid = lax.axis_index("c")
        sid = lax.axis_index("s")
        wid = sid * NC + cid
        base_row = wid * rows_per_worker
        zrows = 125

        zvec = jnp.zeros((LANES,), jnp.float32)

        @pl.loop(0, zrows)
        def _(i):
            for j in range(d // LANES):
                zero_v[i, pl.ds(j * LANES, LANES)] = zvec

        @pl.loop(0, nodes_per_sub // zrows)
        def _(k):
            pltpu.sync_copy(
                zero_v, acc.at[pl.ds(sid * nodes_per_sub + k * zrows, zrows)])

        plsc.subcore_barrier()

        @pl.loop(0, n_chunks)
        def _(g):
            r0 = base_row + g * chunk_rows
            pltpu.sync_copy(row_hbm.at[pl.ds(r0, chunk_rows)], row_v)
            pltpu.sync_copy(col_hbm.at[pl.ds(r0, chunk_rows)], col_v)
            pltpu.sync_copy(a_hbm.at[pl.ds(r0, chunk_rows)], a_v)

            copies = []
            for sb in range(chunk_rows):
                copies.append(pltpu.async_copy(
                    y_hbm.at[row_v.at[sb]],
                    rows_v.at[pl.ds(sb * 128, 128)], sem))
            for cp in copies:
                cp.wait()

            @pl.loop(0, chunk_rows)
            def _(r):
                @pl.loop(0, 128 // LANES)
                def _(q):
                    a16 = a_v[r, pl.ds(q * LANES, LANES)]
                    for l in range(LANES):
                        ab = _broadcast_lane(a16, l)
                        e = r * 128 + q * LANES + l
                        for j in range(d // LANES):
                            sl = pl.ds(j * LANES, LANES)
                            rows_v[e, sl] = rows_v[e, sl] * ab

            for sb in range(chunk_rows):
                pltpu.sync_copy(rows_v.at[pl.ds(sb * 128, 128)],
                                acc.at[col_v.at[sb]], add=True)

        plsc.subcore_barrier()

        @pl.loop(0, nodes_per_sub // zrows)
        def _(k):
            o0 = sid * nodes_per_sub + k * zrows
            pltpu.sync_copy(acc.at[pl.ds(o0, zrows)],
                            out_hbm.at[cid].at[pl.ds(o0, zrows)])

    return sc_kernel


def kernel(x, edge_index_ray, edge_attr_ray, W_t, b_t, W_p, b_p):
    n, d = x.shape
    e = edge_index_ray.shape[1]

    e_rows = e // 128
    e_pad_rows = ((e_rows + NW - 1) // NW) * NW
    pad_rows = e_pad_rows - e_rows

    row = edge_index_ray[0].reshape(e_rows, 128)
    col = edge_index_ray[1].reshape(e_rows, 128)
    dist = edge_attr_ray[:, 0].reshape(e_rows, 128)
    blocking = edge_attr_ray[:, 1].reshape(e_rows, 128)

    nb = 10
    bn = n // nb
    y = pl.pallas_call(
        _prep_body,
        grid=(nb,),
        in_specs=[
            pl.BlockSpec((bn, d), lambda i: (i, 0)),
            pl.BlockSpec((d, d), lambda i: (0, 0)),
            pl.BlockSpec((1, d), lambda i: (0, 0)),
            pl.BlockSpec((1, d), lambda i: (0, 0)),
            pl.BlockSpec((1, d), lambda i: (0, 0)),
        ],
        out_specs=pl.BlockSpec((bn, d), lambda i: (i, 0)),
        out_shape=jax.ShapeDtypeStruct((n, d), jnp.float32),
    )(x, W_p, b_p.reshape(1, d), W_t.reshape(1, d),
      jnp.broadcast_to(b_t.reshape(1, 1), (1, d)))

    a = pl.pallas_call(
        _edge_weight_body,
        out_shape=jax.ShapeDtypeStruct((e_rows, 128), jnp.float32),
    )(blocking, dist)

    row_p = jnp.pad(row, ((0, pad_rows), (0, 0)))
    col_p = jnp.pad(col, ((0, pad_rows), (0, 0)))
    a_p = jnp.pad(a, ((0, pad_rows), (0, 0)))

    sc = _make_sc_scatter(n, d, e_pad_rows, chunk_rows=4)
    partials = sc(y, row_p, col_p, a_p)

    x_new = pl.pallas_call(
        _final_add_body,
        grid=(nb,),
        in_specs=[pl.BlockSpec((bn, d), lambda i: (i, 0))] * 3,
        out_specs=pl.BlockSpec((bn, d), lambda i: (i, 0)),
        out_shape=jax.ShapeDtypeStruct((n, d), jnp.float32),
    )(x, partials[0], partials[1])
    return x_new


# speedup vs baseline: 2.7353x; 2.7353x over previous
import functools

import jax
import jax.numpy as jnp
from jax import lax
from jax.experimental import pallas as pl
from jax.experimental.pallas import tpu as pltpu
from jax.experimental.pallas import tpu_sc as plsc

NC = 2
NS = 16
NW = NC * NS
LANES = 16


def _prep_body(x_ref, wp_ref, bp_ref, wt_ref, bt_ref, y_ref):
    xb = x_ref[...]
    s = jnp.sum(xb * wt_ref[...], axis=1, keepdims=True) + bt_ref[...]
    t = jax.nn.sigmoid(s)
    xp = jnp.dot(xb, wp_ref[...], preferred_element_type=jnp.float32)
    y_ref[...] = (xp + bp_ref[...]) * t


def _make_edge_weight_body(half_n):
    def body(blk_ref, dist_ref, col_ref, a0_ref, a1_ref, c0_ref, c1_ref):
        a = 1.0 / (1.0 + blk_ref[...] + 0.1 * dist_ref[...])
        col = col_ref[...]
        local0 = col < half_n
        a0_ref[...] = jnp.where(local0, a, 0.0)
        a1_ref[...] = jnp.where(local0, 0.0, a)
        c0_ref[...] = jnp.where(local0, col, 0)
        c1_ref[...] = jnp.clip(col - half_n, 0, half_n - 1)
    return body


def _final_add_body(x_ref, p_ref, o_ref):
    o_ref[...] = x_ref[...] + p_ref[...]


def _broadcast_lane(vec, lane):
    idx = jnp.full((LANES, 1), lane, dtype=jnp.int32)
    dnums = lax.GatherDimensionNumbers(
        offset_dims=(), collapsed_slice_dims=(0,), start_index_map=(0,))
    return lax.gather(vec, idx, dnums, slice_sizes=(1,),
                      mode=lax.GatherScatterMode.PROMISE_IN_BOUNDS)


def _make_sc_scatter(half_n, d, e_pad_rows):
    group_rows = 8
    half_rows = 4
    rows_per_worker = e_pad_rows // NS
    n_groups = rows_per_worker // group_rows
    half_edges = half_rows * 128
    nodes_per_sub = half_n // NS
    zrows = 64
    mesh = plsc.VectorSubcoreMesh(core_axis_name="c", subcore_axis_name="s")

    @functools.partial(
        pl.kernel,
        mesh=mesh,
        out_type=jax.ShapeDtypeStruct((NC, half_n, d), jnp.float32),
        scratch_types=[
            pltpu.VMEM((group_rows, 128), jnp.int32),
            pltpu.VMEM((group_rows, 128), jnp.int32),
            pltpu.VMEM((group_rows, 128), jnp.float32),
            pltpu.VMEM((half_edges, d), jnp.float32),
            pltpu.VMEM((zrows, d), jnp.float32),
            pltpu.VMEM_SHARED((half_n, d), jnp.float32),
            pltpu.SemaphoreType.DMA,
        ],
    )
    def sc_kernel(y_hbm, row_hbm, col2_hbm, a2_hbm, out_hbm,
                  row_v, col_v, a_v, rows_v, zero_v, acc, sem):
        cid = lax.axis_index("c")
        sid = lax.axis_index("s")
        col_hbm = col2_hbm.at[cid]
        a_hbm = a2_hbm.at[cid]
        base_row = sid * rows_per_worker

        zvec = jnp.zeros((LANES,), jnp.float32)

        @pl.loop(0, zrows)
        def _(i):
            for j in range(d // LANES):
                zero_v[i, pl.ds(j * LANES, LANES)] = zvec

        @pl.loop(0, nodes_per_sub // zrows)
        def _(k):
            pltpu.sync_copy(
                zero_v, acc.at[pl.ds(sid * nodes_per_sub + k * zrows, zrows)])

        plsc.subcore_barrier()

        @pl.loop(0, n_groups)
        def _(g):
            r0 = base_row + g * group_rows
            pltpu.sync_copy(row_hbm.at[pl.ds(r0, group_rows)], row_v)
            pltpu.sync_copy(col_hbm.at[pl.ds(r0, group_rows)], col_v)
            pltpu.sync_copy(a_hbm.at[pl.ds(r0, group_rows)], a_v)

            for half in range(2):
                copies = []
                for sb in range(half_rows):
                    copies.append(pltpu.async_copy(
                        y_hbm.at[row_v.at[half * half_rows + sb]],
                        rows_v.at[pl.ds(sb * 128, 128)], sem))
                for cp in copies:
                    cp.wait()

                @pl.loop(0, half_rows)
                def _(r):
                    @pl.loop(0, 128 // LANES)
                    def _(q):
                        a16 = a_v[half * half_rows + r,
                                  pl.ds(q * LANES, LANES)]
                        for l in range(LANES):
                            ab = _broadcast_lane(a16, l)
                            e = r * 128 + q * LANES + l
                            for j in range(d // LANES):
                                sl = pl.ds(j * LANES, LANES)
                                rows_v[e, sl] = rows_v[e, sl] * ab

                for sb in range(half_rows):
                    pltpu.sync_copy(
                        rows_v.at[pl.ds(sb * 128, 128)],
                        acc.at[col_v.at[half * half_rows + sb]], add=True)

        plsc.subcore_barrier()

        @pl.loop(0, nodes_per_sub // zrows)
        def _(k):
            o0 = sid * nodes_per_sub + k * zrows
            pltpu.sync_copy(acc.at[pl.ds(o0, zrows)],
                            out_hbm.at[cid].at[pl.ds(o0, zrows)])

    return sc_kernel


def kernel(x, edge_index_ray, edge_attr_ray, W_t, b_t, W_p, b_p):
    n, d = x.shape
    e = edge_index_ray.shape[1]

    e_rows = e // 128
    e_pad_rows = ((e_rows + 8 * NW - 1) // (8 * NW)) * (8 * NW)
    pad_rows = e_pad_rows - e_rows
    n_pad = ((n + 128 * NS - 1) // (128 * NS)) * (128 * NS)

    row = edge_index_ray[0].reshape(e_rows, 128)
    col = edge_index_ray[1].reshape(e_rows, 128)
    dist = edge_attr_ray[:, 0].reshape(e_rows, 128)
    blocking = edge_attr_ray[:, 1].reshape(e_rows, 128)

    nb = 10
    bn = n // nb
    y = pl.pallas_call(
        _prep_body,
        grid=(nb,),
        in_specs=[
            pl.BlockSpec((bn, d), lambda i: (i, 0)),
            pl.BlockSpec((d, d), lambda i: (0, 0)),
            pl.BlockSpec((1, d), lambda i: (0, 0)),
            pl.BlockSpec((1, d), lambda i: (0, 0)),
            pl.BlockSpec((1, d), lambda i: (0, 0)),
        ],
        out_specs=pl.BlockSpec((bn, d), lambda i: (i, 0)),
        out_shape=jax.ShapeDtypeStruct((n, d), jnp.float32),
    )(x, W_p, b_p.reshape(1, d), W_t.reshape(1, d),
      jnp.broadcast_to(b_t.reshape(1, 1), (1, d)))

    half_n = n_pad // 2
    fshape = jax.ShapeDtypeStruct((e_rows, 128), jnp.float32)
    ishape = jax.ShapeDtypeStruct((e_rows, 128), jnp.int32)
    a0, a1, c0, c1 = pl.pallas_call(
        _make_edge_weight_body(half_n),
        out_shape=(fshape, fshape, ishape, ishape),
    )(blocking, dist, col)

    pad = ((0, pad_rows), (0, 0))
    row_p = jnp.pad(row, pad)
    col2 = jnp.stack([jnp.pad(c0, pad), jnp.pad(c1, pad)])
    a2 = jnp.stack([jnp.pad(a0, pad), jnp.pad(a1, pad)])

    sc = _make_sc_scatter(half_n, d, e_pad_rows)
    partials = sc(y, row_p, col2, a2)

    x_new = pl.pallas_call(
        _final_add_body,
        grid=(nb,),
        in_specs=[
            pl.BlockSpec((bn, d), lambda i: (i, 0)),
            pl.BlockSpec((bn, d), lambda i: (i, 0)),
        ],
        out_specs=pl.BlockSpec((bn, d), lambda i: (i, 0)),
        out_shape=jax.ShapeDtypeStruct((n, d), jnp.float32),
    )(x, partials.reshape(2 * half_n, d)[:n])
    return x_new
